# baseline (speedup 1.0000x reference)
import jax
import jax.numpy as jnp
from jax import lax
from jax.experimental import pallas as pl
from jax.experimental.pallas import tpu as pltpu
from jax.experimental.pallas import tpu_sc as plsc

NUM_SEG = 1024
ACC_W = 80
NC, NS = 2, 16
NW = NC * NS
CHUNK = 512
BKH = 8000


def _gate_body(x_ref, wt_ref, wb_ref, b1_ref, w2_ref, b2_ref,
               ge_ref, go_ref, gmax_ref):
    i = pl.program_id(0)
    x = x_ref[...]
    he = jnp.maximum(x @ wt_ref[...] + b1_ref[...], 0.0)
    ho = jnp.maximum(x @ wb_ref[...] + b1_ref[...], 0.0)
    ge = jnp.sum(he * w2_ref[...], axis=1) + b2_ref[0, 0]
    go = jnp.sum(ho * w2_ref[...], axis=1) + b2_ref[0, 0]
    ge_ref[...] = ge.reshape(1, 1, BKH)
    go_ref[...] = go.reshape(1, 1, BKH)
    bm = jnp.maximum(jnp.max(ge), jnp.max(go))

    @pl.when(i == 0)
    def _():
        gmax_ref[...] = jnp.full((1, 1), bm)

    @pl.when(i > 0)
    def _():
        gmax_ref[...] = jnp.maximum(gmax_ref[...], bm)


def _gate(xr, wtop, wbot, gb1, gW2, gb2):
    nh = xr.shape[0]
    nblk = nh // BKH
    return pl.pallas_call(
        _gate_body,
        grid=(nblk,),
        in_specs=[
            pl.BlockSpec((BKH, 128), lambda i: (i, 0)),
            pl.BlockSpec((128, 32), lambda i: (0, 0)),
            pl.BlockSpec((128, 32), lambda i: (0, 0)),
            pl.BlockSpec((1, 32), lambda i: (0, 0)),
            pl.BlockSpec((1, 32), lambda i: (0, 0)),
            pl.BlockSpec((1, 1), lambda i: (0, 0)),
        ],
        out_specs=[
            pl.BlockSpec((1, 1, BKH), lambda i: (i, 0, 0)),
            pl.BlockSpec((1, 1, BKH), lambda i: (i, 0, 0)),
            pl.BlockSpec((1, 1), lambda i: (0, 0)),
        ],
        out_shape=[
            jax.ShapeDtypeStruct((nblk, 1, BKH), jnp.float32),
            jax.ShapeDtypeStruct((nblk, 1, BKH), jnp.float32),
            jax.ShapeDtypeStruct((1, 1), jnp.float32),
        ],
    )(xr, wtop, wbot, gb1, gW2, gb2)


def _sc_pool(xflat, gepad, gopad, spad, gmax16, n):
    rows_per_tile = n // NW
    nchunk = rows_per_tile // CHUNK
    tail = rows_per_tile - nchunk * CHUNK
    tailpad = -(-tail // 16) * 16 if tail else 0
    tail_groups = tail // 16
    tail_rem = tail - tail_groups * 16
    half = CHUNK // 2
    tail_half = tailpad // 2
    glen_main = half + 8
    glen_tail = -(-tail_half // 16) * 16 + 8 if tail else 0

    mesh = plsc.VectorSubcoreMesh(
        core_axis_name="c", subcore_axis_name="s", num_cores=NC, num_subcores=NS
    )

    def body(
        x_hbm, ge_hbm, go_hbm, s_hbm, gmax_hbm, out_hbm,
        xbuf, gebuf, gobuf, sbuf, ebuf, gmv, acc, semx, sem_ge, sem_go, sems,
    ):
        wid = lax.axis_index("c") * NS + lax.axis_index("s")
        base = wid * rows_per_tile
        iota = lax.iota(jnp.int32, 16)
        onehot0 = (iota == 0).astype(jnp.float32)
        zf = jnp.zeros((16,), jnp.float32)
        zi = jnp.zeros((16,), jnp.int32)

        pltpu.sync_copy(gmax_hbm, gmv)

        def zbody(i, _):
            for d in range(ACC_W // 16):
                acc[i, pl.ds(16 * d, 16)] = zf
            return 0

        lax.fori_loop(0, NUM_SEG, zbody, 0)

        def flush(a, cur):
            for d in range(4):
                plsc.addupdate_scatter(acc, [cur, iota + 16 * d], a[d])
            plsc.addupdate_scatter(acc, [cur, iota + 64], a[4] * onehot0)

        def direct_row(i, epos):
            e_b = plsc.load_gather(ebuf, [jnp.full((16,), 1, jnp.int32) * epos])
            s_b = plsc.load_gather(sbuf, [jnp.full((16,), 1, jnp.int32) * i])
            rb = pl.multiple_of(i * 64, 64)
            for d in range(4):
                xv = xbuf[pl.ds(rb + 16 * d, 16)]
                plsc.addupdate_scatter(acc, [s_b, iota + 16 * d], xv * e_b)
            plsc.addupdate_scatter(acc, [s_b, iota + 64], e_b * onehot0)

        def group_body(gi, carry):
            a0, a1, a2, a3, ad, cur = carry
            g0 = pl.multiple_of(gi * 16, 16)
            g2 = gi * 8
            seg16 = sbuf[pl.ds(g0, 16)]
            allsame = jnp.min((seg16 == cur).astype(jnp.int32))

            def fast(_):
                r0, r1, r2, r3, rd = a0, a1, a2, a3, ad
                for r in range(16):
                    epos = 256 * (r & 1) + g2 + (r >> 1)
                    idxc = jnp.full((16,), 1, jnp.int32) * epos
                    e_b = plsc.load_gather(ebuf, [idxc])
                    rbase = pl.multiple_of((g0 + r) * 64, 64)
                    r0 = r0 + xbuf[pl.ds(rbase, 16)] * e_b
                    r1 = r1 + xbuf[pl.ds(rbase + 16, 16)] * e_b
                    r2 = r2 + xbuf[pl.ds(rbase + 32, 16)] * e_b
                    r3 = r3 + xbuf[pl.ds(rbase + 48, 16)] * e_b
                    rd = rd + e_b
                return (r0, r1, r2, r3, rd, cur)

            def slow(_):
                flush((a0, a1, a2, a3, ad), cur)
                for r in range(16):
                    direct_row(g0 + r, 256 * (r & 1) + g2 + (r >> 1))
                lastc = jnp.full((16,), 1, jnp.int32) * (g0 + 15)
                newcur = plsc.load_gather(sbuf, [lastc])
                return (zf, zf, zf, zf, zf, newcur)

            return lax.cond(allsame == 1, fast, slow, 0)

        def load_chunk(row0, nrows, npad, glen):
            erow0 = row0 // 2
            off = lax.rem(erow0, 8)
            e0 = pl.multiple_of(erow0 - off, 8)
            cx = pltpu.async_copy(
                x_hbm.at[pl.ds(pl.multiple_of(row0 * 64, 64), nrows * 64)],
                xbuf.at[pl.ds(0, nrows * 64)],
                semx,
            )
            cg = pltpu.async_copy(
                ge_hbm.at[pl.ds(e0, glen)], gebuf.at[pl.ds(0, glen)], sem_ge
            )
            co = pltpu.async_copy(
                go_hbm.at[pl.ds(e0, glen)], gobuf.at[pl.ds(0, glen)], sem_go
            )
            cs = pltpu.async_copy(
                s_hbm.at[pl.ds(pl.multiple_of(row0, 8), npad)],
                sbuf.at[pl.ds(0, npad)],
                sems,
            )
            cx.wait()
            cg.wait()
            co.wait()
            cs.wait()
            return off

        def prepass(off, hcount):
            iota = lax.iota(jnp.int32, 16)
            for k in range(0, -(-hcount // 16) * 16, 16):
                gv = plsc.load_gather(gebuf, [off + k + iota])
                ebuf[pl.ds(k, 16)] = jnp.exp(gv - gmv[...])
                gv = plsc.load_gather(gobuf, [off + k + iota])
                ebuf[pl.ds(half + k, 16)] = jnp.exp(gv - gmv[...])

        def chunk_body(c, carry):
            off = load_chunk(base + c * CHUNK, CHUNK, CHUNK, glen_main)
            prepass(off, half)
            return lax.fori_loop(0, CHUNK // 16, group_body, carry)

        carry = (zf, zf, zf, zf, zf, zi)
        carry = lax.fori_loop(0, nchunk, chunk_body, carry)

        if tail:
            off = load_chunk(base + nchunk * CHUNK, tail, tailpad, glen_tail)
            prepass(off, tail_half)
            carry = lax.fori_loop(0, tail_groups, group_body, carry)
            for r in range(tail_rem):
                i = tail_groups * 16 + r
                direct_row(i, 256 * (r & 1) + tail_groups * 8 + (r >> 1))

        flush(carry[:5], carry[5])

        pltpu.sync_copy(acc, out_hbm.at[wid])

    run = pl.kernel(
        body,
        out_type=jax.ShapeDtypeStruct((NW, NUM_SEG, ACC_W), jnp.float32),
        mesh=mesh,
        compiler_params=pltpu.CompilerParams(
            use_tc_tiling_on_sc=False, needs_layout_passes=False
        ),
        scratch_types=[
            pltpu.VMEM((CHUNK * 64,), jnp.float32),
            pltpu.VMEM((glen_main,), jnp.float32),
            pltpu.VMEM((glen_main,), jnp.float32),
            pltpu.VMEM((CHUNK,), jnp.int32),
            pltpu.VMEM((CHUNK,), jnp.float32),
            pltpu.VMEM((16,), jnp.float32),
            pltpu.VMEM((NUM_SEG, ACC_W), jnp.float32),
            pltpu.SemaphoreType.DMA,
            pltpu.SemaphoreType.DMA,
            pltpu.SemaphoreType.DMA,
            pltpu.SemaphoreType.DMA,
        ],
    )
    return run(xflat, gepad, gopad, spad, gmax16)


def _final_body(p_ref, w1_ref, b1_ref, w2_ref, b2_ref, out_ref):
    s = jnp.sum(p_ref[...], axis=0)
    num = s[:, :64]
    den = s[:, 64:65]
    hg = num / (den + 1e-16)
    h = jnp.maximum(hg @ w1_ref[...] + b1_ref[...], 0.0)
    out_ref[...] = h @ w2_ref[...] + b2_ref[...]


def _final(partials, mW1, mb1, mW2, mb2):
    return pl.pallas_call(
        _final_body,
        out_shape=jax.ShapeDtypeStruct((NUM_SEG, 1), jnp.float32),
    )(partials, mW1, mb1, mW2, mb2)


@jax.jit
def kernel(x, batch, gW1, gb1, gW2, gb2, mW1, mb1, mW2, mb2):
    n = x.shape[0]
    xflat = x.reshape(-1)
    xr = xflat.reshape(n // 2, 128)
    zpad = jnp.zeros((64, 32), jnp.float32)
    wtop = jnp.concatenate([gW1, zpad], axis=0)
    wbot = jnp.concatenate([zpad, gW1], axis=0)
    ge, go, gmax = _gate(
        xr, wtop, wbot, gb1.reshape(1, -1), gW2.reshape(1, -1), gb2.reshape(1, -1)
    )
    seg = batch.astype(jnp.int32)
    gepad = jnp.concatenate([ge.reshape(-1), jnp.zeros((64,), jnp.float32)])
    gopad = jnp.concatenate([go.reshape(-1), jnp.zeros((64,), jnp.float32)])
    spad = jnp.concatenate([seg, jnp.zeros((16,), jnp.int32)])
    gmax16 = jnp.broadcast_to(gmax.reshape(1), (16,))
    partials = _sc_pool(xflat, gepad, gopad, spad, gmax16, n)
    return _final(partials, mW1, mb1.reshape(1, -1), mW2, mb2.reshape(1, -1))


# speedup vs baseline: 1.8508x; 1.8508x over previous
import jax
import jax.numpy as jnp
from jax import lax
from jax.experimental import pallas as pl
from jax.experimental.pallas import tpu as pltpu
from jax.experimental.pallas import tpu_sc as plsc

NUM_SEG = 1024
ACC_W = 80
NC, NS = 2, 16
NW = NC * NS
CHUNK = 512
BK = 16000


def _gate_body(xt_ref, w1_ref, b1_ref, w2_ref, b2_ref, g_ref, gmax_ref):
    i = pl.program_id(0)
    ht = lax.dot_general(
        w1_ref[...], xt_ref[...], (((0,), (0,)), ((), ()))
    )
    h = jnp.maximum(ht + b1_ref[...], 0.0)
    g = jnp.sum(h * w2_ref[...], axis=0) + b2_ref[0, 0]
    g_ref[...] = g.reshape(1, 1, BK)
    bm = jnp.max(g)

    @pl.when(i == 0)
    def _():
        gmax_ref[...] = jnp.full((1, 1), bm)

    @pl.when(i > 0)
    def _():
        gmax_ref[...] = jnp.maximum(gmax_ref[...], bm)


def _gate(xt, gW1, gb1, gW2, gb2):
    n = xt.shape[1]
    nblk = n // BK
    return pl.pallas_call(
        _gate_body,
        grid=(nblk,),
        in_specs=[
            pl.BlockSpec((64, BK), lambda i: (0, i)),
            pl.BlockSpec((64, 32), lambda i: (0, 0)),
            pl.BlockSpec((32, 1), lambda i: (0, 0)),
            pl.BlockSpec((32, 1), lambda i: (0, 0)),
            pl.BlockSpec((1, 1), lambda i: (0, 0)),
        ],
        out_specs=[
            pl.BlockSpec((1, 1, BK), lambda i: (i, 0, 0)),
            pl.BlockSpec((1, 1), lambda i: (0, 0)),
        ],
        out_shape=[
            jax.ShapeDtypeStruct((nblk, 1, BK), jnp.float32),
            jax.ShapeDtypeStruct((1, 1), jnp.float32),
        ],
    )(xt, gW1, gb1, gW2, gb2)


def _sc_pool(xflat, gpad, spad, gmax16, n):
    rows_per_tile = n // NW
    nchunk = rows_per_tile // CHUNK
    tail = rows_per_tile - nchunk * CHUNK
    tailpad = -(-tail // 16) * 16 if tail else 0
    tail_groups = tail // 16
    tail_rem = tail - tail_groups * 16

    mesh = plsc.VectorSubcoreMesh(
        core_axis_name="c", subcore_axis_name="s", num_cores=NC, num_subcores=NS
    )

    def body(
        x_hbm, g_hbm, s_hbm, gmax_hbm, out_hbm,
        xbuf, gbuf, sbuf, ebuf, gmv, acc, semx, semg, sems,
    ):
        wid = lax.axis_index("c") * NS + lax.axis_index("s")
        base = wid * rows_per_tile
        iota = lax.iota(jnp.int32, 16)
        onehot0 = (iota == 0).astype(jnp.float32)
        zf = jnp.zeros((16,), jnp.float32)
        zi = jnp.zeros((16,), jnp.int32)

        pltpu.sync_copy(gmax_hbm, gmv)

        def zbody(i, _):
            for d in range(ACC_W // 16):
                acc[i, pl.ds(16 * d, 16)] = zf
            return 0

        lax.fori_loop(0, NUM_SEG, zbody, 0)

        def flush(a, cur):
            for d in range(4):
                plsc.addupdate_scatter(acc, [cur, iota + 16 * d], a[d])
            plsc.addupdate_scatter(acc, [cur, iota + 64], a[4] * onehot0)

        def direct_row(i):
            e_b = plsc.load_gather(ebuf, [jnp.full((16,), 1, jnp.int32) * i])
            s_b = plsc.load_gather(sbuf, [jnp.full((16,), 1, jnp.int32) * i])
            rb = pl.multiple_of(i * 64, 64)
            for d in range(4):
                xv = xbuf[pl.ds(rb + 16 * d, 16)]
                plsc.addupdate_scatter(acc, [s_b, iota + 16 * d], xv * e_b)
            plsc.addupdate_scatter(acc, [s_b, iota + 64], e_b * onehot0)

        def group_body(gi, carry):
            a0, a1, a2, a3, ad, cur = carry
            g0 = pl.multiple_of(gi * 16, 16)
            seg16 = sbuf[pl.ds(g0, 16)]
            allsame = jnp.min((seg16 == cur).astype(jnp.int32))

            def fast(_):
                r0, r1, r2, r3, rd = a0, a1, a2, a3, ad
                for r in range(16):
                    idxc = jnp.full((16,), 1, jnp.int32) * (g0 + r)
                    e_b = plsc.load_gather(ebuf, [idxc])
                    rbase = pl.multiple_of((g0 + r) * 64, 64)
                    r0 = r0 + xbuf[pl.ds(rbase, 16)] * e_b
                    r1 = r1 + xbuf[pl.ds(rbase + 16, 16)] * e_b
                    r2 = r2 + xbuf[pl.ds(rbase + 32, 16)] * e_b
                    r3 = r3 + xbuf[pl.ds(rbase + 48, 16)] * e_b
                    rd = rd + e_b
                return (r0, r1, r2, r3, rd, cur)

            def slow(_):
                flush((a0, a1, a2, a3, ad), cur)
                for r in range(16):
                    direct_row(g0 + r)
                lastc = jnp.full((16,), 1, jnp.int32) * (g0 + 15)
                newcur = plsc.load_gather(sbuf, [lastc])
                return (zf, zf, zf, zf, zf, newcur)

            return lax.cond(allsame == 1, fast, slow, 0)

        def load_chunk(row0, nrows, npad):
            cx = pltpu.async_copy(
                x_hbm.at[pl.ds(pl.multiple_of(row0 * 64, 64), nrows * 64)],
                xbuf.at[pl.ds(0, nrows * 64)],
                semx,
            )
            r8 = pl.multiple_of(row0, 8)
            cg = pltpu.async_copy(
                g_hbm.at[pl.ds(r8, npad)], gbuf.at[pl.ds(0, npad)], semg
            )
            cs = pltpu.async_copy(
                s_hbm.at[pl.ds(r8, npad)], sbuf.at[pl.ds(0, npad)], sems
            )
            cx.wait()
            cg.wait()
            cs.wait()

        def prepass(count):
            for k in range(0, count, 16):
                ebuf[pl.ds(k, 16)] = jnp.exp(gbuf[pl.ds(k, 16)] - gmv[...])

        def chunk_body(c, carry):
            load_chunk(base + c * CHUNK, CHUNK, CHUNK)
            prepass(CHUNK)
            return lax.fori_loop(0, CHUNK // 16, group_body, carry)

        carry = (zf, zf, zf, zf, zf, zi)
        carry = lax.fori_loop(0, nchunk, chunk_body, carry)

        if tail:
            load_chunk(base + nchunk * CHUNK, tail, tailpad)
            prepass(tailpad)
            carry = lax.fori_loop(0, tail_groups, group_body, carry)
            for r in range(tail_rem):
                direct_row(tail_groups * 16 + r)

        flush(carry[:5], carry[5])

        pltpu.sync_copy(acc, out_hbm.at[wid])

    run = pl.kernel(
        body,
        out_type=jax.ShapeDtypeStruct((NW, NUM_SEG, ACC_W), jnp.float32),
        mesh=mesh,
        compiler_params=pltpu.CompilerParams(
            use_tc_tiling_on_sc=False, needs_layout_passes=False
        ),
        scratch_types=[
            pltpu.VMEM((CHUNK * 64,), jnp.float32),
            pltpu.VMEM((CHUNK,), jnp.float32),
            pltpu.VMEM((CHUNK,), jnp.int32),
            pltpu.VMEM((CHUNK,), jnp.float32),
            pltpu.VMEM((16,), jnp.float32),
            pltpu.VMEM((NUM_SEG, ACC_W), jnp.float32),
            pltpu.SemaphoreType.DMA,
            pltpu.SemaphoreType.DMA,
            pltpu.SemaphoreType.DMA,
        ],
    )
    return run(xflat, gpad, spad, gmax16)


def _final_body(p_ref, w1_ref, b1_ref, w2_ref, b2_ref, out_ref):
    s = jnp.sum(p_ref[...], axis=0)
    num = s[:, :64]
    den = s[:, 64:65]
    hg = num / (den + 1e-16)
    h = jnp.maximum(hg @ w1_ref[...] + b1_ref[...], 0.0)
    out_ref[...] = h @ w2_ref[...] + b2_ref[...]


def _final(partials, mW1, mb1, mW2, mb2):
    return pl.pallas_call(
        _final_body,
        out_shape=jax.ShapeDtypeStruct((NUM_SEG, 1), jnp.float32),
    )(partials, mW1, mb1, mW2, mb2)


@jax.jit
def kernel(x, batch, gW1, gb1, gW2, gb2, mW1, mb1, mW2, mb2):
    n = x.shape[0]
    xt = x.T
    xflat = x.reshape(-1)
    g3, gmax = _gate(xt, gW1, gb1.reshape(-1, 1), gW2, gb2.reshape(1, 1))
    seg = batch.astype(jnp.int32)
    gpad = jnp.concatenate([g3.reshape(-1), jnp.zeros((16,), jnp.float32)])
    spad = jnp.concatenate([seg, jnp.zeros((16,), jnp.int32)])
    gmax16 = jnp.broadcast_to(gmax.reshape(1), (16,))
    partials = _sc_pool(xflat, gpad, spad, gmax16, n)
    return _final(partials, mW1, mb1.reshape(1, -1), mW2, mb2.reshape(1, -1))
